# baseline (speedup 1.0000x reference)
import jax
import jax.numpy as jnp
from jax.experimental import pallas as pl
from jax.experimental.pallas import tpu as pltpu


def _ffn_body(eid_ref, xs_ref, w1_ref, b1_ref, w2_ref, b2_ref, y_ref):
    xv = xs_ref[...]
    h = jnp.dot(xv, w1_ref[0].astype(jnp.bfloat16),
                preferred_element_type=jnp.float32)
    h = jnp.maximum(h + b1_ref[0, 0], 0.0)
    y = jnp.dot(h.astype(jnp.bfloat16), w2_ref[0].astype(jnp.bfloat16),
                preferred_element_type=jnp.float32)
    y_ref[...] = y + b2_ref[0, 0]


def _ffn(xs, tile_eid, W1, b1, W2, b2, T):
    NPAD, D = xs.shape
    E, _, H = W1.shape
    NT = NPAD // T
    grid_spec = pltpu.PrefetchScalarGridSpec(
        num_scalar_prefetch=1,
        grid=(NT,),
        in_specs=[
            pl.BlockSpec((T, D), lambda i, eid: (i, 0)),
            pl.BlockSpec((1, D, H), lambda i, eid: (eid[i], 0, 0)),
            pl.BlockSpec((1, 1, H), lambda i, eid: (eid[i], 0, 0)),
            pl.BlockSpec((1, H, D), lambda i, eid: (eid[i], 0, 0)),
            pl.BlockSpec((1, 1, D), lambda i, eid: (eid[i], 0, 0)),
        ],
        out_specs=pl.BlockSpec((T, D), lambda i, eid: (i, 0)),
    )
    return pl.pallas_call(
        _ffn_body,
        grid_spec=grid_spec,
        out_shape=jax.ShapeDtypeStruct((NPAD, D), jnp.float32),
    )(tile_eid, xs, W1, b1.reshape(E, 1, H), W2, b2.reshape(E, 1, D))


def _gate_body(x_ref, wg_ref, s_ref, xb_ref):
    xv = x_ref[...]
    s_ref[...] = jnp.dot(xv, wg_ref[...], preferred_element_type=jnp.float32)
    xb_ref[...] = xv.astype(jnp.bfloat16)


def _gate(xf, Wg, bg):
    N, D = xf.shape
    E = Wg.shape[1]
    TG = 512
    wg_pad = jnp.zeros((D, 128), Wg.dtype).at[:, :E].set(Wg)
    scores, xb = pl.pallas_call(
        _gate_body,
        grid=(N // TG,),
        in_specs=[
            pl.BlockSpec((TG, D), lambda i: (i, 0)),
            pl.BlockSpec((D, 128), lambda i: (0, 0)),
        ],
        out_specs=[
            pl.BlockSpec((TG, 128), lambda i: (i, 0)),
            pl.BlockSpec((TG, D), lambda i: (i, 0)),
        ],
        out_shape=[
            jax.ShapeDtypeStruct((N, 128), jnp.float32),
            jax.ShapeDtypeStruct((N, D), jnp.bfloat16),
        ],
    )(xf, wg_pad)
    return scores[:, :E] + bg, xb


def kernel(x, Wg, bg, W1, b1, W2, b2):
    B, S, D = x.shape
    E = Wg.shape[1]
    N = B * S
    T = 256
    NT = N // T + E
    NPAD = NT * T

    xf = x.reshape(N, D)
    scores, xb = _gate(xf, Wg, bg)
    eid = jnp.argmax(scores, axis=1).astype(jnp.int32)

    oh = (eid[:, None] == jnp.arange(E, dtype=jnp.int32)[None, :]).astype(jnp.int32)
    cc = jnp.cumsum(oh, axis=0)
    counts = cc[-1]
    rank = jnp.take_along_axis(cc, eid[:, None], axis=1)[:, 0] - 1
    tiles_per_e = (counts + T - 1) // T
    tile_base = jnp.concatenate([jnp.zeros((1,), jnp.int32),
                                 jnp.cumsum(tiles_per_e)[:-1].astype(jnp.int32)])
    pad_off = tile_base * T
    pos = pad_off[eid] + rank
    perm = jnp.full((NPAD,), N, dtype=jnp.int32).at[pos].set(
        jnp.arange(N, dtype=jnp.int32))
    tile_eid = jnp.zeros((NT,), jnp.int32).at[tile_base].max(
        jnp.arange(E, dtype=jnp.int32))
    tile_eid = jax.lax.cummax(tile_eid)
    used_tiles = tile_base[-1] + tiles_per_e[-1]
    tile_eid = jnp.where(jnp.arange(NT) < used_tiles, tile_eid, 0)

    xpad = jnp.concatenate([xb, jnp.zeros((1, D), xb.dtype)], axis=0)
    xs = xpad[perm]
    ys = _ffn(xs, tile_eid, W1, b1, W2, b2, T)
    out = jnp.zeros((N + 1, D), x.dtype).at[perm].set(ys, mode='drop')
    out = out[:N].reshape(B, S, D)
    return (out, scores.reshape(B, S, E))
